# Initial kernel scaffold; baseline (speedup 1.0000x reference)
import jax
import jax.numpy as jnp
from jax.experimental import pallas as pl


def kernel(x_local, x_virtual, edge_index_ll, edge_index_lv, edge_index_vv, batch_local, W_ll_self, W_ll_nbr, b_ll, W_lv_self, W_lv_nbr, b_lv, W_vv_self, W_vv_nbr, b_vv, lin1_W, lin1_b, lin2_W, lin2_b):
    raise NotImplementedError("write your pallas kernel here")


# speedup vs baseline: 2.5879x; 2.5879x over previous
import jax
import jax.numpy as jnp
from jax import lax
from jax.experimental import pallas as pl
from jax.experimental.pallas import tpu as pltpu
from jax.experimental.pallas import tpu_sc as plsc

NL = 10000
NV = 1000
NVP = 1008
H = 256
HH = 128
G = 16
NCLS = 10
E_LL, E_LV, E_VV = 160000, 10000, 16000
CH_LL, CH_LV, CH_VV = 79, 5, 8
ACC_LL, ACC_V = 10112, 1024
STR_LL, STR_V = ACC_LL // 16, ACC_V // 16

_mesh = plsc.VectorSubcoreMesh(core_axis_name="c", subcore_axis_name="s")
_f32 = jnp.float32
_i32 = jnp.int32


def _agg_body(xl2, xv2, sll, dll, slv, dlv, svv, dvv, zeros,
              out_ll, out_lv, out_vv,
              acc_ll, acc_lv, acc_vv,
              src_c, dst_c, idx_v, rows_v, sem):
    cid = lax.axis_index("c")
    sid = lax.axis_index("s")
    pltpu.sync_copy(zeros.at[pl.ds(sid * STR_LL, STR_LL)],
                    acc_ll.at[pl.ds(sid * STR_LL, STR_LL)])
    pltpu.sync_copy(zeros.at[pl.ds(sid * STR_V, STR_V)],
                    acc_lv.at[pl.ds(sid * STR_V, STR_V)])
    pltpu.sync_copy(zeros.at[pl.ds(sid * STR_V, STR_V)],
                    acc_vv.at[pl.ds(sid * STR_V, STR_V)])
    plsc.subcore_barrier()

    def rel(n_chunks, src_h, dst_h, table, acc):
        def body(j, carry):
            pltpu.sync_copy(src_h.at[sid, pl.ds(j * 128, 128)], src_c)
            pltpu.sync_copy(dst_h.at[sid, j], dst_c)
            for k in range(8):
                s = src_c[pl.ds(k * 16, 16)]
                idx_v[pl.ds(k * 16, 16)] = s * 2 + cid
            pltpu.async_copy(table.at[idx_v], rows_v, sem).wait()
            pltpu.sync_copy(rows_v, acc.at[dst_c], add=True)
            return carry
        lax.fori_loop(0, n_chunks, body, 0)

    rel(CH_LL, sll, dll, xl2, acc_ll)
    rel(CH_LV, slv, dlv, xl2, acc_lv)
    rel(CH_VV, svv, dvv, xv2, acc_vv)
    plsc.subcore_barrier()
    pltpu.sync_copy(acc_ll.at[pl.ds(sid * STR_LL, STR_LL)],
                    out_ll.at[cid, pl.ds(sid * STR_LL, STR_LL)])
    pltpu.sync_copy(acc_lv.at[pl.ds(sid * STR_V, STR_V)],
                    out_lv.at[cid, pl.ds(sid * STR_V, STR_V)])
    pltpu.sync_copy(acc_vv.at[pl.ds(sid * STR_V, STR_V)],
                    out_vv.at[cid, pl.ds(sid * STR_V, STR_V)])


_agg = pl.kernel(
    _agg_body,
    (jax.ShapeDtypeStruct((2, ACC_LL, HH), _f32),
     jax.ShapeDtypeStruct((2, ACC_V, HH), _f32),
     jax.ShapeDtypeStruct((2, ACC_V, HH), _f32)),
    mesh=_mesh,
    scratch_types=[
        pltpu.VMEM_SHARED((ACC_LL, HH), _f32),
        pltpu.VMEM_SHARED((ACC_V, HH), _f32),
        pltpu.VMEM_SHARED((ACC_V, HH), _f32),
        pltpu.VMEM((128,), _i32),
        pltpu.VMEM((128,), _i32),
        pltpu.VMEM((128,), _i32),
        pltpu.VMEM((128, HH), _f32),
        pltpu.SemaphoreType.DMA,
    ],
)


def _cnt_body(dll, dlv, dvv, ones_h, zeros,
              out_ll, out_lv, out_vv,
              c_ll, c_lv, c_vv, dst_c, ones_v):
    cid = lax.axis_index("c")
    sid = lax.axis_index("s")
    pltpu.sync_copy(ones_h, ones_v)

    def rel(n_chunks, dst_h, c_acc):
        def body(j, carry):
            pltpu.sync_copy(dst_h.at[sid, j], dst_c)
            pltpu.sync_copy(ones_v, c_acc.at[dst_c], add=True)
            return carry
        lax.fori_loop(0, n_chunks, body, 0)

    @pl.when(cid == 0)
    def _():
        pltpu.sync_copy(zeros.at[pl.ds(sid * STR_LL, STR_LL)],
                        c_ll.at[pl.ds(sid * STR_LL, STR_LL)])
        plsc.subcore_barrier()
        rel(CH_LL, dll, c_ll)
        plsc.subcore_barrier()
        pltpu.sync_copy(c_ll.at[pl.ds(sid * STR_LL, STR_LL)],
                        out_ll.at[pl.ds(sid * STR_LL, STR_LL)])

    @pl.when(cid == 1)
    def _():
        pltpu.sync_copy(zeros.at[pl.ds(sid * STR_V, STR_V)],
                        c_lv.at[pl.ds(sid * STR_V, STR_V)])
        pltpu.sync_copy(zeros.at[pl.ds(sid * STR_V, STR_V)],
                        c_vv.at[pl.ds(sid * STR_V, STR_V)])
        plsc.subcore_barrier()
        rel(CH_LV, dlv, c_lv)
        rel(CH_VV, dvv, c_vv)
        plsc.subcore_barrier()
        pltpu.sync_copy(c_lv.at[pl.ds(sid * STR_V, STR_V)],
                        out_lv.at[pl.ds(sid * STR_V, STR_V)])
        pltpu.sync_copy(c_vv.at[pl.ds(sid * STR_V, STR_V)],
                        out_vv.at[pl.ds(sid * STR_V, STR_V)])


_counts = pl.kernel(
    _cnt_body,
    (jax.ShapeDtypeStruct((ACC_LL, HH), _f32),
     jax.ShapeDtypeStruct((ACC_V, HH), _f32),
     jax.ShapeDtypeStruct((ACC_V, HH), _f32)),
    mesh=_mesh,
    scratch_types=[
        pltpu.VMEM_SHARED((ACC_LL, HH), _f32),
        pltpu.VMEM_SHARED((ACC_V, HH), _f32),
        pltpu.VMEM_SHARED((ACC_V, HH), _f32),
        pltpu.VMEM((128,), _i32),
        pltpu.VMEM((128, HH), _f32),
    ],
)


def _tcl_body(x, a0, a1, cnt, ws, wn, b, o):
    inv = 1.0 / jnp.maximum(cnt[:, 0:1], 1.0)
    wnv = wn[...]
    acc = jnp.dot(x[...], ws[...], preferred_element_type=_f32)
    acc += jnp.dot(a0[...] * inv, wnv[0:HH, :], preferred_element_type=_f32)
    acc += jnp.dot(a1[...] * inv, wnv[HH:H, :], preferred_element_type=_f32)
    o[...] = jnp.maximum(acc + b[...], 0.0)


_tc_local = pl.pallas_call(
    _tcl_body,
    out_shape=jax.ShapeDtypeStruct((NL, H), _f32),
    grid=(10,),
    in_specs=[
        pl.BlockSpec((1000, H), lambda i: (i, 0)),
        pl.BlockSpec((1000, HH), lambda i: (i, 0)),
        pl.BlockSpec((1000, HH), lambda i: (i, 0)),
        pl.BlockSpec((1000, HH), lambda i: (i, 0)),
        pl.BlockSpec((H, H), lambda i: (0, 0)),
        pl.BlockSpec((H, H), lambda i: (0, 0)),
        pl.BlockSpec((1, H), lambda i: (0, 0)),
    ],
    out_specs=pl.BlockSpec((1000, H), lambda i: (i, 0)),
)


def _tcv_body(xv, alv0, alv1, avv0, avv1, clv, cvv,
              ws_lv, wn_lv, ws_vv, wn_vv, blv, bvv, o):
    inv_lv = 1.0 / jnp.maximum(clv[:, 0:1], 1.0)
    inv_vv = 1.0 / jnp.maximum(cvv[:, 0:1], 1.0)
    xvv = xv[...]
    wnl = wn_lv[...]
    wnv = wn_vv[...]
    acc = jnp.dot(xvv, ws_lv[...], preferred_element_type=_f32)
    acc += jnp.dot(xvv, ws_vv[...], preferred_element_type=_f32)
    acc += jnp.dot(alv0[...] * inv_lv, wnl[0:HH, :], preferred_element_type=_f32)
    acc += jnp.dot(alv1[...] * inv_lv, wnl[HH:H, :], preferred_element_type=_f32)
    acc += jnp.dot(avv0[...] * inv_vv, wnv[0:HH, :], preferred_element_type=_f32)
    acc += jnp.dot(avv1[...] * inv_vv, wnv[HH:H, :], preferred_element_type=_f32)
    o[...] = jnp.maximum(acc + blv[...] + bvv[...], 0.0)


_tc_virtual = pl.pallas_call(
    _tcv_body,
    out_shape=jax.ShapeDtypeStruct((NVP, H), _f32),
)


def _pool_body(x, bb, w1, b1, w2, b2, o, acc, cacc):
    i = pl.program_id(0)

    @pl.when(i == 0)
    def _():
        acc[...] = jnp.zeros_like(acc)
        cacc[...] = jnp.zeros_like(cacc)

    oh = (bb[...] == lax.broadcasted_iota(_i32, (1000, G), 1)).astype(_f32)
    acc[...] += lax.dot_general(oh, x[...], (((0,), (0,)), ((), ())),
                                preferred_element_type=_f32)
    cacc[...] += lax.dot_general(oh, jnp.ones((1000, HH), _f32),
                                 (((0,), (0,)), ((), ())),
                                 preferred_element_type=_f32)

    @pl.when(i == 9)
    def _():
        pooled = acc[...] / jnp.maximum(cacc[...][:, 0:1], 1.0)
        h = jnp.maximum(
            jnp.dot(pooled, w1[...], preferred_element_type=_f32) + b1[...], 0.0)
        o[...] = jnp.dot(h, w2[...], preferred_element_type=_f32) + b2[...]


_pool = pl.pallas_call(
    _pool_body,
    out_shape=jax.ShapeDtypeStruct((G, HH), _f32),
    grid=(10,),
    in_specs=[
        pl.BlockSpec((1000, H), lambda i: (i, 0)),
        pl.BlockSpec((1000, G), lambda i: (i, 0)),
        pl.BlockSpec((H, H), lambda i: (0, 0)),
        pl.BlockSpec((1, H), lambda i: (0, 0)),
        pl.BlockSpec((H, HH), lambda i: (0, 0)),
        pl.BlockSpec((1, HH), lambda i: (0, 0)),
    ],
    out_specs=pl.BlockSpec((G, HH), lambda i: (0, 0)),
    scratch_shapes=[
        pltpu.VMEM((G, H), _f32),
        pltpu.VMEM((G, HH), _f32),
    ],
)


def _prep_edges(ei, n_edges, n_chunks, dummy):
    pad = 16 * n_chunks * 128 - n_edges
    src = jnp.concatenate([ei[0].astype(_i32), jnp.zeros((pad,), _i32)])
    dst = jnp.concatenate([ei[1].astype(_i32), jnp.full((pad,), dummy, _i32)])
    return src.reshape(16, n_chunks * 128), dst.reshape(16, n_chunks, 128)


def kernel(x_local, x_virtual, edge_index_ll, edge_index_lv, edge_index_vv,
           batch_local, W_ll_self, W_ll_nbr, b_ll, W_lv_self, W_lv_nbr, b_lv,
           W_vv_self, W_vv_nbr, b_vv, lin1_W, lin1_b, lin2_W, lin2_b):
    x = x_local.astype(_f32)
    xv = jnp.zeros((NVP, H), _f32).at[:NV].set(x_virtual.astype(_f32))
    sll, dll = _prep_edges(edge_index_ll, E_LL, CH_LL, NL)
    slv, dlv = _prep_edges(edge_index_lv, E_LV, CH_LV, NV)
    svv, dvv = _prep_edges(edge_index_vv, E_VV, CH_VV, NV)
    zeros = jnp.zeros((ACC_LL, HH), _f32)
    ones128 = jnp.ones((128, HH), _f32)

    cll, clv, cvv = _counts(dll, dlv, dvv, ones128, zeros)
    cll, clv, cvv = cll[:NL], clv[:NVP], cvv[:NVP]

    for l in range(3):
        a_ll, a_lv, a_vv = _agg(x.reshape(2 * NL, HH), xv.reshape(2 * NVP, HH),
                                sll, dll, slv, dlv, svv, dvv, zeros)
        x = _tc_local(x, a_ll[0, :NL], a_ll[1, :NL], cll,
                      W_ll_self[l], W_ll_nbr[l], b_ll[l][None, :])
        xv = _tc_virtual(xv, a_lv[0, :NVP], a_lv[1, :NVP],
                         a_vv[0, :NVP], a_vv[1, :NVP], clv, cvv,
                         W_lv_self[l], W_lv_nbr[l], W_vv_self[l], W_vv_nbr[l],
                         b_lv[l][None, :], b_vv[l][None, :])

    bb = jnp.broadcast_to(batch_local.astype(_i32)[:, None], (NL, G))
    w2 = jnp.zeros((H, HH), _f32).at[:, :NCLS].set(lin2_W)
    b2 = jnp.zeros((1, HH), _f32).at[0, :NCLS].set(lin2_b)
    out = _pool(x, bb, lin1_W, lin1_b[None, :], w2, b2)
    return out[:, :NCLS]
